# baseline (speedup 1.0000x reference)
import functools

import jax
import jax.numpy as jnp
from jax.experimental import pallas as pl
from jax.experimental.pallas import tpu as pltpu

N_NODE = 50000
C = 128
H = 2
EPS = 1e-5

_BM = 1000


def _proj_body(x_ref, w_ref, att_src_ref, att_dst_ref, h_ref, a_ref):
    h = jnp.dot(x_ref[...], w_ref[...], preferred_element_type=jnp.float32)
    h_ref[...] = h
    hh = h.reshape(-1, H, C)
    a = (hh * att_dst_ref[...]).sum(-1)
    b = (hh * att_src_ref[...]).sum(-1)
    a_ref[...] = jnp.concatenate([b, a], axis=-1)


def _project(x, W, att_src, att_dst):
    n = x.shape[0]
    grid = (n // _BM,)
    h, a = pl.pallas_call(
        _proj_body,
        grid=grid,
        in_specs=[
            pl.BlockSpec((_BM, C), lambda i: (i, 0)),
            pl.BlockSpec((C, H * C), lambda i: (0, 0)),
            pl.BlockSpec((1, H, C), lambda i: (0, 0, 0)),
            pl.BlockSpec((1, H, C), lambda i: (0, 0, 0)),
        ],
        out_specs=[
            pl.BlockSpec((_BM, H * C), lambda i: (i, 0)),
            pl.BlockSpec((_BM, 2 * H), lambda i: (i, 0)),
        ],
        out_shape=[
            jax.ShapeDtypeStruct((n, H * C), jnp.float32),
            jax.ShapeDtypeStruct((n, 2 * H), jnp.float32),
        ],
    )(x, W, att_src, att_dst)
    return h, a


def _gat(x_src, x_dst, edge_index, W, att_src, att_dst, bias, num_dst):
    src, dst = edge_index[0], edge_index[1]
    h_src, a_s = _project(x_src, W, att_src, att_dst)
    _, a_d = _project(x_dst, W, att_src, att_dst)
    a_src = a_s[:, :H]
    a_dst = a_d[:, H:]
    alpha = a_src[src] + a_dst[dst]
    alpha = jax.nn.leaky_relu(alpha, 0.2)
    amax = jax.ops.segment_max(alpha, dst, num_segments=num_dst)
    amax = jnp.where(jnp.isfinite(amax), amax, 0.0)
    ex = jnp.exp(alpha - amax[dst])
    denom = jax.ops.segment_sum(ex, dst, num_segments=num_dst)
    w = ex / (denom[dst] + 1e-16)
    msg = h_src.reshape(-1, H, C)[src] * w[:, :, None]
    out = jax.ops.segment_sum(msg, dst, num_segments=num_dst)
    return out.mean(axis=1) + bias


def _ln_relu_body(x_ref, w_ref, b_ref, o_ref):
    x = x_ref[...]
    mu = x.mean(axis=-1, keepdims=True)
    var = ((x - mu) ** 2).mean(axis=-1, keepdims=True)
    y = (x - mu) * jax.lax.rsqrt(var + EPS) * w_ref[...] + b_ref[...]
    o_ref[...] = jnp.maximum(y, 0.0)


def _ln_relu(x, w, b):
    n = x.shape[0]
    return pl.pallas_call(
        _ln_relu_body,
        grid=(n // _BM,),
        in_specs=[
            pl.BlockSpec((_BM, C), lambda i: (i, 0)),
            pl.BlockSpec((1, C), lambda i: (0, 0)),
            pl.BlockSpec((1, C), lambda i: (0, 0)),
        ],
        out_specs=pl.BlockSpec((_BM, C), lambda i: (i, 0)),
        out_shape=jax.ShapeDtypeStruct((n, C), jnp.float32),
    )(x, w.reshape(1, C), b.reshape(1, C))


def kernel(x_user, x_item, edge_index_ui, edge_index_iu, W_ui, att_src_ui,
           att_dst_ui, bias_ui, W_iu, att_src_iu, att_dst_iu, bias_iu,
           ln_w_user, ln_b_user, ln_w_item, ln_b_item):
    item_out = _gat(x_user, x_item, edge_index_ui, W_ui, att_src_ui,
                    att_dst_ui, bias_ui, x_item.shape[0])
    user_out = _gat(x_item, x_user, edge_index_iu, W_iu, att_src_iu,
                    att_dst_iu, bias_iu, x_user.shape[0])
    user_out = _ln_relu(user_out, ln_w_user, ln_b_user)
    item_out = _ln_relu(item_out, ln_w_item, ln_b_item)
    return (user_out, item_out)
